# baseline (speedup 1.0000x reference)
import functools

import jax
import jax.numpy as jnp
from jax import lax
from jax.experimental import pallas as pl
from jax.experimental.pallas import tpu as pltpu
from jax.experimental.pallas import tpu_sc as plsc

_BIN_THRESH = float(2.0 ** -24)
_N_IMG = 16
_H = 512
_W = 512
_VALID = _H - 2
_POS_PER_HIST = _N_IMG * _VALID * _VALID

_N_TILES = 32
_PACK = 176
_ROWS = _PACK // 2
_CHUNK = _ROWS * _W
_HALF_OFF = 528 * 16
_HIST_WORDS = 2 * _HALF_OFF
_CNT_HALF = 640
_CNT_ROW = 2 * _CNT_HALF


def _codes_body(src, dst):
    if True:
        x = src[0]
        xb = (x > _BIN_THRESH).astype(jnp.int32)
        rc = (xb[:, 0:510] << 2) + (xb[:, 1:511] << 1) + xb[:, 2:512]
        code = (rc[0:510] << 6) + (rc[1:511] << 3) + rc[2:512]
        code = jnp.concatenate(
            [code, jnp.full((_VALID, 2), 512, jnp.int32)], axis=1)
        code = jnp.concatenate(
            [code, jnp.full((3 * _PACK - _VALID, _W), 512, jnp.int32)],
            axis=0)
        dst[0] = (code[0:_PACK] | (code[_PACK:2 * _PACK] << 10)
                  | (code[2 * _PACK:3 * _PACK] << 20))


def _codes(x):
    return pl.pallas_call(
        _codes_body,
        grid=(_N_IMG,),
        in_specs=[pl.BlockSpec((1, _H, _W), lambda i: (i, 0, 0))],
        out_specs=pl.BlockSpec((1, _PACK, _W), lambda i: (i, 0, 0)),
        out_shape=jax.ShapeDtypeStruct((_N_IMG, _PACK, _W), jnp.int32),
    )(x)


def _hist_body(cin, ctgt, out_hbm, buf, hist, counts, sem0, sem1):
    wid = lax.axis_index("s") * 2 + lax.axis_index("c")
    lane = lax.iota(jnp.int32, 16)
    ones = jnp.ones((16,), jnp.float32)

    @plsc.parallel_loop(0, _HIST_WORDS // 16, unroll=8)
    def _zero(i):
        hist[pl.ds(i * 16, 16)] = jnp.zeros((16,), jnp.float32)

    img = wid >> 1
    r0 = (wid & 1) * _ROWS
    chunks = [(half, src, dr, nr)
              for half, src in ((0, cin), (1, ctgt))
              for dr, nr in ((0, 48), (48, 40))]
    sems = (sem0, sem1)
    n = len(chunks)
    _, src0, dr0, nr0 = chunks[0]
    pending = pltpu.async_copy(
        src0.at[img, pl.ds(r0 + dr0, nr0), :],
        buf.at[0, pl.ds(0, nr0)], sems[0])
    for ci in range(n):
        half, _, _, nrows = chunks[ci]
        s = ci % 2
        if ci + 1 < n:
            _, nsrc, ndr, nnr = chunks[ci + 1]
            nxt = pltpu.async_copy(
                nsrc.at[img, pl.ds(r0 + ndr, nnr), :],
                buf.at[1 - s, pl.ds(0, nnr)], sems[1 - s])
        pending.wait()

        @plsc.parallel_loop(0, nrows * (_W // 16), unroll=16)
        def _chunk(j, _off=half * _HALF_OFF, _s=s):
            r = j >> 5
            c = (j & 31) << 4
            w = buf[_s, r, pl.ds(c, 16)]
            lane_off = lane + _off
            idx0 = ((w << 4) & 0x3FF0) + lane_off
            plsc.addupdate_scatter(hist, [idx0], ones)
            idx1 = (lax.shift_right_logical(w, 6) & 0x3FF0) + lane_off
            plsc.addupdate_scatter(hist, [idx1], ones)
            idx2 = (lax.shift_right_logical(w, 16) & 0x3FF0) + lane_off
            plsc.addupdate_scatter(hist, [idx2], ones)

        if ci + 1 < n:
            pending = nxt

    for half in range(2):
        hoff = half * _HALF_OFF
        coff = half * _CNT_HALF

        def red_body(g, _):
            addr0 = hoff + ((g * 16 + lane) << 4)
            acc = jnp.zeros((16,), jnp.float32)
            for l in range(16):
                acc = acc + plsc.load_gather(hist, [addr0 + l])
            counts[pl.ds(coff + g * 16, 16)] = acc
            return 0

        lax.fori_loop(0, 33, red_body, 0)

    pltpu.sync_copy(counts, out_hbm.at[wid])


@functools.cache
def _hist():
    return pl.kernel(
        _hist_body,
        out_type=jax.ShapeDtypeStruct((_N_TILES, _CNT_ROW), jnp.float32),
        mesh=plsc.VectorSubcoreMesh(core_axis_name="c", subcore_axis_name="s"),
        compiler_params=pltpu.CompilerParams(needs_layout_passes=False),
        scratch_types=[
            pltpu.VMEM((2, 48, _W), jnp.int32),
            pltpu.VMEM((_HIST_WORDS,), jnp.float32),
            pltpu.VMEM((_CNT_ROW,), jnp.float32),
            pltpu.SemaphoreType.DMA,
            pltpu.SemaphoreType.DMA,
        ],
    )

_MSE_SCALE = 1.0 / (float(_POS_PER_HIST) ** 2 * 512.0 * float(_N_IMG))


def _mse_body(p_ref, out_ref):
    s = jnp.sum(p_ref[...], axis=0, keepdims=True)
    d = s[:, 0:512] - s[:, _CNT_HALF:_CNT_HALF + 512]
    out_ref[0, 0] = jnp.sum(d * d) * _MSE_SCALE


def _mse(parts):
    return pl.pallas_call(
        _mse_body,
        out_specs=pl.BlockSpec(memory_space=pltpu.SMEM),
        out_shape=jax.ShapeDtypeStruct((1, 1), jnp.float32),
    )(parts)


def kernel(input, target):
    inp = input.reshape(_N_IMG, _H, _W)
    tgt = target.reshape(_N_IMG, _H, _W)
    cin = _codes(inp)
    ctgt = _codes(tgt)
    parts = _hist()(cin, ctgt)
    return _mse(parts)[0, 0]


# speedup vs baseline: 1.1458x; 1.1458x over previous
import functools

import jax
import jax.numpy as jnp
from jax import lax
from jax.experimental import pallas as pl
from jax.experimental.pallas import tpu as pltpu
from jax.experimental.pallas import tpu_sc as plsc

_BIN_THRESH = float(2.0 ** -24)
_N_IMG = 16
_H = 512
_W = 512
_VALID = _H - 2
_POS_PER_HIST = _N_IMG * _VALID * _VALID

_N_TILES = 32
_PACK = 176
_ROWS = _PACK // 2
_CHUNK = _ROWS * _W
_HALF_OFF = 528 * 16
_HIST_WORDS = 2 * _HALF_OFF
_CNT_HALF = 640
_CNT_ROW = 2 * _CNT_HALF


def _codes_body(inp_ref, tgt_ref, cin_ref, ctgt_ref):
    for src, dst in ((inp_ref, cin_ref), (tgt_ref, ctgt_ref)):
        x = src[0]
        xb = (x > _BIN_THRESH).astype(jnp.int32)
        rc = (xb[:, 0:510] << 2) + (xb[:, 1:511] << 1) + xb[:, 2:512]
        code = (rc[0:510] << 6) + (rc[1:511] << 3) + rc[2:512]
        code = jnp.concatenate(
            [code, jnp.full((_VALID, 2), 512, jnp.int32)], axis=1)
        code = jnp.concatenate(
            [code, jnp.full((3 * _PACK - _VALID, _W), 512, jnp.int32)],
            axis=0)
        dst[0] = (code[0:_PACK] | (code[_PACK:2 * _PACK] << 10)
                  | (code[2 * _PACK:3 * _PACK] << 20))


def _codes(inp, tgt):
    return pl.pallas_call(
        _codes_body,
        grid=(_N_IMG,),
        in_specs=[
            pl.BlockSpec((1, _H, _W), lambda i: (i, 0, 0)),
            pl.BlockSpec((1, _H, _W), lambda i: (i, 0, 0)),
        ],
        out_specs=[
            pl.BlockSpec((1, _PACK, _W), lambda i: (i, 0, 0)),
            pl.BlockSpec((1, _PACK, _W), lambda i: (i, 0, 0)),
        ],
        out_shape=[
            jax.ShapeDtypeStruct((_N_IMG, _PACK, _W), jnp.int32),
            jax.ShapeDtypeStruct((_N_IMG, _PACK, _W), jnp.int32),
        ],
    )(inp, tgt)


def _hist_body(cin, ctgt, out_hbm, buf, hist, counts, sem0, sem1):
    wid = lax.axis_index("s") * 2 + lax.axis_index("c")
    lane = lax.iota(jnp.int32, 16)
    ones = jnp.ones((16,), jnp.float32)

    @plsc.parallel_loop(0, _HIST_WORDS // 16, unroll=8)
    def _zero(i):
        hist[pl.ds(i * 16, 16)] = jnp.zeros((16,), jnp.float32)

    img = wid >> 1
    r0 = (wid & 1) * _ROWS
    chunks = [(half, src, dr, nr)
              for half, src in ((0, cin), (1, ctgt))
              for dr, nr in ((0, 48), (48, 40))]
    sems = (sem0, sem1)
    n = len(chunks)
    _, src0, dr0, nr0 = chunks[0]
    pending = pltpu.async_copy(
        src0.at[img, pl.ds(r0 + dr0, nr0), :],
        buf.at[0, pl.ds(0, nr0)], sems[0])
    for ci in range(n):
        half, _, _, nrows = chunks[ci]
        s = ci % 2
        if ci + 1 < n:
            _, nsrc, ndr, nnr = chunks[ci + 1]
            nxt = pltpu.async_copy(
                nsrc.at[img, pl.ds(r0 + ndr, nnr), :],
                buf.at[1 - s, pl.ds(0, nnr)], sems[1 - s])
        pending.wait()

        @plsc.parallel_loop(0, nrows * (_W // 16), unroll=16)
        def _chunk(j, _off=half * _HALF_OFF, _s=s):
            r = j >> 5
            c = (j & 31) << 4
            w = buf[_s, r, pl.ds(c, 16)]
            lane_off = lane + _off
            idx0 = ((w << 4) & 0x3FF0) + lane_off
            plsc.addupdate_scatter(hist, [idx0], ones)
            idx1 = (lax.shift_right_logical(w, 6) & 0x3FF0) + lane_off
            plsc.addupdate_scatter(hist, [idx1], ones)
            idx2 = (lax.shift_right_logical(w, 16) & 0x3FF0) + lane_off
            plsc.addupdate_scatter(hist, [idx2], ones)

        if ci + 1 < n:
            pending = nxt

    for half in range(2):
        hoff = half * _HALF_OFF
        coff = half * _CNT_HALF

        def red_body(g, _):
            addr0 = hoff + ((g * 16 + lane) << 4)
            acc = jnp.zeros((16,), jnp.float32)
            for l in range(16):
                acc = acc + plsc.load_gather(hist, [addr0 + l])
            counts[pl.ds(coff + g * 16, 16)] = acc
            return 0

        lax.fori_loop(0, 33, red_body, 0)

    pltpu.sync_copy(counts, out_hbm.at[wid])


@functools.cache
def _hist():
    return pl.kernel(
        _hist_body,
        out_type=jax.ShapeDtypeStruct((_N_TILES, _CNT_ROW), jnp.float32),
        mesh=plsc.VectorSubcoreMesh(core_axis_name="c", subcore_axis_name="s"),
        compiler_params=pltpu.CompilerParams(needs_layout_passes=False),
        scratch_types=[
            pltpu.VMEM((2, 48, _W), jnp.int32),
            pltpu.VMEM((_HIST_WORDS,), jnp.float32),
            pltpu.VMEM((_CNT_ROW,), jnp.float32),
            pltpu.SemaphoreType.DMA,
            pltpu.SemaphoreType.DMA,
        ],
    )

_MSE_SCALE = 1.0 / (float(_POS_PER_HIST) ** 2 * 512.0 * float(_N_IMG))


def _mse_body(p_ref, out_ref):
    s = jnp.sum(p_ref[...], axis=0, keepdims=True)
    d = s[:, 0:512] - s[:, _CNT_HALF:_CNT_HALF + 512]
    out_ref[0, 0] = jnp.sum(d * d) * _MSE_SCALE


def _mse(parts):
    return pl.pallas_call(
        _mse_body,
        out_specs=pl.BlockSpec(memory_space=pltpu.SMEM),
        out_shape=jax.ShapeDtypeStruct((1, 1), jnp.float32),
    )(parts)


def kernel(input, target):
    inp = input.reshape(_N_IMG, _H, _W)
    tgt = target.reshape(_N_IMG, _H, _W)
    cin, ctgt = _codes(inp, tgt)
    parts = _hist()(cin, ctgt)
    return _mse(parts)[0, 0]


# speedup vs baseline: 1.1460x; 1.0001x over previous
import functools

import jax
import jax.numpy as jnp
from jax import lax
from jax.experimental import pallas as pl
from jax.experimental.pallas import tpu as pltpu
from jax.experimental.pallas import tpu_sc as plsc

_BIN_THRESH = float(2.0 ** -24)
_N_IMG = 16
_H = 512
_W = 512
_VALID = _H - 2
_POS_PER_HIST = _N_IMG * _VALID * _VALID

_N_TILES = 32
_PACK = 176
_ROWS = _PACK // 2
_CHUNK = _ROWS * _W
_HALF_OFF = 528 * 16
_HIST_WORDS = 2 * _HALF_OFF
_CNT_HALF = 640
_CNT_ROW = 2 * _CNT_HALF


def _codes_body(inp_ref, tgt_ref, cin_ref, ctgt_ref):
    for src, dst in ((inp_ref, cin_ref), (tgt_ref, ctgt_ref)):
        x = src[0]
        xb = (x > _BIN_THRESH).astype(jnp.int32)
        rc = (xb[:, 0:510] << 2) + (xb[:, 1:511] << 1) + xb[:, 2:512]
        code = (rc[0:510] << 6) + (rc[1:511] << 3) + rc[2:512]
        code = jnp.concatenate(
            [code, jnp.full((3 * _PACK - _VALID, _VALID), 512, jnp.int32)],
            axis=0)
        packed = (code[0:_PACK] | (code[_PACK:2 * _PACK] << 10)
                  | (code[2 * _PACK:3 * _PACK] << 20))
        dst[0, :, 0:_VALID] = packed


def _codes(inp, tgt):
    return pl.pallas_call(
        _codes_body,
        grid=(_N_IMG,),
        in_specs=[
            pl.BlockSpec((1, _H, _W), lambda i: (i, 0, 0)),
            pl.BlockSpec((1, _H, _W), lambda i: (i, 0, 0)),
        ],
        out_specs=[
            pl.BlockSpec((1, _PACK, _W), lambda i: (i, 0, 0)),
            pl.BlockSpec((1, _PACK, _W), lambda i: (i, 0, 0)),
        ],
        out_shape=[
            jax.ShapeDtypeStruct((_N_IMG, _PACK, _W), jnp.int32),
            jax.ShapeDtypeStruct((_N_IMG, _PACK, _W), jnp.int32),
        ],
    )(inp, tgt)


def _hist_body(cin, ctgt, out_hbm, buf, hist, counts, sem0, sem1):
    wid = lax.axis_index("s") * 2 + lax.axis_index("c")
    lane = lax.iota(jnp.int32, 16)
    ones = jnp.ones((16,), jnp.float32)

    @plsc.parallel_loop(0, _HIST_WORDS // 16, unroll=8)
    def _zero(i):
        hist[pl.ds(i * 16, 16)] = jnp.zeros((16,), jnp.float32)

    img = wid >> 1
    r0 = (wid & 1) * _ROWS
    chunks = [(half, src, dr, nr)
              for half, src in ((0, cin), (1, ctgt))
              for dr, nr in ((0, 48), (48, 40))]
    sems = (sem0, sem1)
    n = len(chunks)
    _, src0, dr0, nr0 = chunks[0]
    pending = pltpu.async_copy(
        src0.at[img, pl.ds(r0 + dr0, nr0), :],
        buf.at[0, pl.ds(0, nr0)], sems[0])
    for ci in range(n):
        half, _, _, nrows = chunks[ci]
        s = ci % 2
        if ci + 1 < n:
            _, nsrc, ndr, nnr = chunks[ci + 1]
            nxt = pltpu.async_copy(
                nsrc.at[img, pl.ds(r0 + ndr, nnr), :],
                buf.at[1 - s, pl.ds(0, nnr)], sems[1 - s])
        pending.wait()

        @plsc.parallel_loop(0, nrows * (_W // 16), unroll=16)
        def _chunk(j, _off=half * _HALF_OFF, _s=s):
            g = j & 31
            r = j >> 5
            c = g << 4
            w = buf[_s, r, pl.ds(c, 16)]
            m = lane < jnp.where(g == 31, 14, 16)
            lane_off = lane + _off
            idx0 = ((w << 4) & 0x3FF0) + lane_off
            plsc.addupdate_scatter(hist, [idx0], ones, mask=m)
            idx1 = (lax.shift_right_logical(w, 6) & 0x3FF0) + lane_off
            plsc.addupdate_scatter(hist, [idx1], ones, mask=m)
            idx2 = (lax.shift_right_logical(w, 16) & 0x3FF0) + lane_off
            plsc.addupdate_scatter(hist, [idx2], ones, mask=m)

        if ci + 1 < n:
            pending = nxt

    for half in range(2):
        hoff = half * _HALF_OFF
        coff = half * _CNT_HALF

        def red_body(g, _):
            addr0 = hoff + ((g * 16 + lane) << 4)
            acc = jnp.zeros((16,), jnp.float32)
            for l in range(16):
                acc = acc + plsc.load_gather(hist, [addr0 + l])
            counts[pl.ds(coff + g * 16, 16)] = acc
            return 0

        lax.fori_loop(0, 33, red_body, 0)

    pltpu.sync_copy(counts, out_hbm.at[wid])


@functools.cache
def _hist():
    return pl.kernel(
        _hist_body,
        out_type=jax.ShapeDtypeStruct((_N_TILES, _CNT_ROW), jnp.float32),
        mesh=plsc.VectorSubcoreMesh(core_axis_name="c", subcore_axis_name="s"),
        compiler_params=pltpu.CompilerParams(needs_layout_passes=False),
        scratch_types=[
            pltpu.VMEM((2, 48, _W), jnp.int32),
            pltpu.VMEM((_HIST_WORDS,), jnp.float32),
            pltpu.VMEM((_CNT_ROW,), jnp.float32),
            pltpu.SemaphoreType.DMA,
            pltpu.SemaphoreType.DMA,
        ],
    )

_MSE_SCALE = 1.0 / (float(_POS_PER_HIST) ** 2 * 512.0 * float(_N_IMG))


def _mse_body(p_ref, out_ref):
    s = jnp.sum(p_ref[...], axis=0, keepdims=True)
    d = s[:, 0:512] - s[:, _CNT_HALF:_CNT_HALF + 512]
    out_ref[0, 0] = jnp.sum(d * d) * _MSE_SCALE


def _mse(parts):
    return pl.pallas_call(
        _mse_body,
        out_specs=pl.BlockSpec(memory_space=pltpu.SMEM),
        out_shape=jax.ShapeDtypeStruct((1, 1), jnp.float32),
    )(parts)


def kernel(input, target):
    inp = input.reshape(_N_IMG, _H, _W)
    tgt = target.reshape(_N_IMG, _H, _W)
    cin, ctgt = _codes(inp, tgt)
    parts = _hist()(cin, ctgt)
    return _mse(parts)[0, 0]


# speedup vs baseline: 1.1464x; 1.0004x over previous
import functools

import jax
import jax.numpy as jnp
from jax import lax
from jax.experimental import pallas as pl
from jax.experimental.pallas import tpu as pltpu
from jax.experimental.pallas import tpu_sc as plsc

_BIN_THRESH = float(2.0 ** -24)
_N_IMG = 16
_H = 512
_W = 512
_VALID = _H - 2
_POS_PER_HIST = _N_IMG * _VALID * _VALID

_N_TILES = 32
_PACK = 176
_ROWS = _PACK // 2
_CHUNK = _ROWS * _W
_HALF_OFF = 528 * 16
_HIST_WORDS = 2 * _HALF_OFF
_CNT_HALF = 640
_CNT_ROW = 2 * _CNT_HALF


def _codes_body(inp_ref, tgt_ref, cin_ref, ctgt_ref):
    for src, dst in ((inp_ref, cin_ref), (tgt_ref, ctgt_ref)):
        x = src[0]
        xb = (x > _BIN_THRESH).astype(jnp.int32)
        rc = (xb[:, 0:510] << 2) + (xb[:, 1:511] << 1) + xb[:, 2:512]
        code = (rc[0:510] << 6) + (rc[1:511] << 3) + rc[2:512]
        code = jnp.concatenate(
            [code, jnp.full((_VALID, 2), 512, jnp.int32)], axis=1)
        code = jnp.concatenate(
            [code, jnp.full((3 * _PACK - _VALID, _W), 512, jnp.int32)],
            axis=0)
        dst[0] = (code[0:_PACK] | (code[_PACK:2 * _PACK] << 10)
                  | (code[2 * _PACK:3 * _PACK] << 20))


def _codes(inp, tgt):
    return pl.pallas_call(
        _codes_body,
        grid=(_N_IMG,),
        in_specs=[
            pl.BlockSpec((1, _H, _W), lambda i: (i, 0, 0)),
            pl.BlockSpec((1, _H, _W), lambda i: (i, 0, 0)),
        ],
        out_specs=[
            pl.BlockSpec((1, _PACK, _W), lambda i: (i, 0, 0)),
            pl.BlockSpec((1, _PACK, _W), lambda i: (i, 0, 0)),
        ],
        out_shape=[
            jax.ShapeDtypeStruct((_N_IMG, _PACK, _W), jnp.int32),
            jax.ShapeDtypeStruct((_N_IMG, _PACK, _W), jnp.int32),
        ],
    )(inp, tgt)


def _hist_body(cin, ctgt, out_hbm, buf, hist, counts, sem0, sem1):
    wid = lax.axis_index("s") * 2 + lax.axis_index("c")
    lane = lax.iota(jnp.int32, 16)
    ones = jnp.ones((16,), jnp.float32)

    @plsc.parallel_loop(0, _HIST_WORDS // 16, unroll=8)
    def _zero(i):
        hist[pl.ds(i * 16, 16)] = jnp.zeros((16,), jnp.float32)

    img = wid >> 1
    r0 = (wid & 1) * _ROWS
    chunks = [(half, src, dr, nr)
              for half, src in ((0, cin), (1, ctgt))
              for dr, nr in ((0, 48), (48, 40))]
    sems = (sem0, sem1)
    n = len(chunks)
    _, src0, dr0, nr0 = chunks[0]
    pending = pltpu.async_copy(
        src0.at[img, pl.ds(r0 + dr0, nr0), :],
        buf.at[0, pl.ds(0, nr0)], sems[0])
    for ci in range(n):
        half, _, _, nrows = chunks[ci]
        s = ci % 2
        if ci + 1 < n:
            _, nsrc, ndr, nnr = chunks[ci + 1]
            nxt = pltpu.async_copy(
                nsrc.at[img, pl.ds(r0 + ndr, nnr), :],
                buf.at[1 - s, pl.ds(0, nnr)], sems[1 - s])
        pending.wait()

        @plsc.parallel_loop(0, nrows * (_W // 16), unroll=16)
        def _chunk(j, _off=half * _HALF_OFF, _s=s):
            r = j >> 5
            c = (j & 31) << 4
            w = buf[_s, r, pl.ds(c, 16)]
            lane_off = lane + _off
            idx0 = ((w << 4) & 0x3FF0) + lane_off
            plsc.addupdate_scatter(hist, [idx0], ones)
            idx1 = (lax.shift_right_logical(w, 6) & 0x3FF0) + lane_off
            plsc.addupdate_scatter(hist, [idx1], ones)
            idx2 = (lax.shift_right_logical(w, 16) & 0x3FF0) + lane_off
            plsc.addupdate_scatter(hist, [idx2], ones)

        if ci + 1 < n:
            pending = nxt

    for half in range(2):
        hoff = half * _HALF_OFF
        coff = half * _CNT_HALF

        def red_body(g, _):
            addr0 = hoff + ((g * 16 + lane) << 4)
            acc = jnp.zeros((16,), jnp.float32)
            for l in range(16):
                acc = acc + plsc.load_gather(hist, [addr0 + l])
            counts[pl.ds(coff + g * 16, 16)] = acc
            return 0

        lax.fori_loop(0, 33, red_body, 0)

    pltpu.sync_copy(counts, out_hbm.at[wid])


@functools.cache
def _hist():
    return pl.kernel(
        _hist_body,
        out_type=jax.ShapeDtypeStruct((_N_TILES, _CNT_ROW), jnp.float32),
        mesh=plsc.VectorSubcoreMesh(core_axis_name="c", subcore_axis_name="s"),
        compiler_params=pltpu.CompilerParams(needs_layout_passes=False),
        scratch_types=[
            pltpu.VMEM((2, 48, _W), jnp.int32),
            pltpu.VMEM((_HIST_WORDS,), jnp.float32),
            pltpu.VMEM((_CNT_ROW,), jnp.float32),
            pltpu.SemaphoreType.DMA,
            pltpu.SemaphoreType.DMA,
        ],
    )

_MSE_SCALE = 1.0 / (float(_POS_PER_HIST) ** 2 * 512.0 * float(_N_IMG))


def _mse_body(p_ref, out_ref):
    s = jnp.sum(p_ref[...], axis=0, keepdims=True)
    d = s[:, 0:512] - s[:, _CNT_HALF:_CNT_HALF + 512]
    out_ref[0, 0] = jnp.sum(d * d) * _MSE_SCALE


def _mse(parts):
    return pl.pallas_call(
        _mse_body,
        out_specs=pl.BlockSpec(memory_space=pltpu.SMEM),
        out_shape=jax.ShapeDtypeStruct((1, 1), jnp.float32),
    )(parts)


def kernel(input, target):
    inp = input.reshape(_N_IMG, _H, _W)
    tgt = target.reshape(_N_IMG, _H, _W)
    cin, ctgt = _codes(inp, tgt)
    parts = _hist()(cin, ctgt)
    return _mse(parts)[0, 0]
